# baseline (speedup 1.0000x reference)
import functools

import jax
import jax.numpy as jnp
from jax import lax
from jax.experimental import pallas as pl
from jax.experimental.pallas import tpu as pltpu
from jax.experimental.pallas import tpu_sc as plsc

_NC = 2
_NS = 16
_NW = _NC * _NS
_CH = 128


def _pe_table(max_len, d_embed):
    pos = jnp.arange(max_len, dtype=jnp.float32)[:, None]
    i = jnp.arange(0, d_embed, 2, dtype=jnp.float32)[None, :]
    angle = pos / jnp.power(10000.0, i / d_embed)
    pe = jnp.zeros((max_len, d_embed), dtype=jnp.float32)
    pe = pe.at[:, 0::2].set(jnp.sin(angle))
    pe = pe.at[:, 1::2].set(jnp.cos(angle))
    return pe


@functools.partial(jax.jit, static_argnums=(3, 4, 5))
def _emb_call(tok_table, idx, pe_ext, N, D, S):
    n_per_w = N // _NW
    n_chunks = n_per_w // _CH
    pe_rows = pe_ext.shape[0]
    mesh = plsc.VectorSubcoreMesh(core_axis_name="c", subcore_axis_name="s")

    @functools.partial(
        pl.kernel,
        out_type=jax.ShapeDtypeStruct((N, D), jnp.float32),
        mesh=mesh,
        scratch_types=[
            pltpu.VMEM((n_per_w,), jnp.int32),
            pltpu.VMEM((_CH, D), jnp.float32),
            pltpu.VMEM((pe_rows, D), jnp.float32),
            pltpu.SemaphoreType.DMA,
        ],
        compiler_params=pltpu.CompilerParams(use_tc_tiling_on_sc=False),
    )
    def emb(tok_hbm, idx_hbm, pe_hbm, out_hbm, idx_v, buf, pe_v, sem):
        wid = lax.axis_index("s") * _NC + lax.axis_index("c")
        base = wid * n_per_w
        pltpu.sync_copy(idx_hbm.at[pl.ds(base, n_per_w)], idx_v)
        pltpu.sync_copy(pe_hbm, pe_v)

        def body(g, _):
            row0 = g * _CH
            pltpu.async_copy(
                tok_hbm.at[idx_v.at[pl.ds(row0, _CH)]], buf, sem
            ).wait()
            start = lax.rem(base + row0, S)

            def radd(r, _):
                p = start + r
                for k in range(D // 16):
                    sl = pl.ds(k * 16, 16)
                    buf[r, sl] = buf[r, sl] + pe_v[p, sl]
                return 0

            lax.fori_loop(0, _CH, radd, 0, unroll=4)
            pltpu.sync_copy(buf, out_hbm.at[pl.ds(base + row0, _CH)])
            return 0

        lax.fori_loop(0, n_chunks, body, 0)

    return emb(tok_table, idx, pe_ext)


def kernel(x, tok_table):
    B, S = x.shape
    V, D = tok_table.shape
    N = B * S
    idx = x.reshape(N).astype(jnp.int32)
    pe = _pe_table(S, D)
    pe_ext = jnp.concatenate([pe, pe[:_CH]], axis=0)
    out = _emb_call(tok_table, idx, pe_ext, N, D, S)
    return out.reshape(B, S, D)


# speedup vs baseline: 1.4423x; 1.4423x over previous
import functools

import jax
import jax.numpy as jnp
from jax import lax
from jax.experimental import pallas as pl
from jax.experimental.pallas import tpu as pltpu
from jax.experimental.pallas import tpu_sc as plsc

_NC = 2
_NS = 16
_NW = _NC * _NS
_CH = 128


def _pe_table(max_len, d_embed):
    pos = jnp.arange(max_len, dtype=jnp.float32)[:, None]
    i = jnp.arange(0, d_embed, 2, dtype=jnp.float32)[None, :]
    angle = pos / jnp.power(10000.0, i / d_embed)
    pe = jnp.zeros((max_len, d_embed), dtype=jnp.float32)
    pe = pe.at[:, 0::2].set(jnp.sin(angle))
    pe = pe.at[:, 1::2].set(jnp.cos(angle))
    return pe


_NBUF = 4


@functools.partial(jax.jit, static_argnums=(3, 4, 5))
def _emb_call(tok_table, idx, pe_ext, N, D, S):
    n_per_w = N // _NW
    n_chunks = n_per_w // _CH
    assert n_chunks % _NBUF == 0
    pe_rows = pe_ext.shape[0]
    mesh = plsc.VectorSubcoreMesh(core_axis_name="c", subcore_axis_name="s")

    @functools.partial(
        pl.kernel,
        out_type=jax.ShapeDtypeStruct((N, D), jnp.float32),
        mesh=mesh,
        scratch_types=[
            pltpu.VMEM((n_per_w,), jnp.int32),
            [pltpu.VMEM((_CH, D), jnp.float32)] * _NBUF,
            pltpu.VMEM((pe_rows, D), jnp.float32),
            [pltpu.SemaphoreType.DMA] * _NBUF,
            [pltpu.SemaphoreType.DMA] * _NBUF,
        ],
        compiler_params=pltpu.CompilerParams(use_tc_tiling_on_sc=False),
    )
    def emb(tok_hbm, idx_hbm, pe_hbm, out_hbm, idx_v, bufs, pe_v, gsems, ssems):
        wid = lax.axis_index("s") * _NC + lax.axis_index("c")
        base = wid * n_per_w
        pltpu.sync_copy(idx_hbm.at[pl.ds(base, n_per_w)], idx_v)
        pltpu.sync_copy(pe_hbm, pe_v)

        def fire_gather(g, bi):
            pltpu.async_copy(
                tok_hbm.at[idx_v.at[pl.ds(g * _CH, _CH)]], bufs[bi], gsems[bi]
            )

        def wait_gather(bi):
            pltpu.make_async_copy(
                tok_hbm.at[idx_v.at[pl.ds(0, _CH)]], bufs[bi], gsems[bi]
            ).wait()

        def wait_store(bi):
            pltpu.make_async_copy(
                bufs[bi], out_hbm.at[pl.ds(0, _CH)], ssems[bi]
            ).wait()

        fire_gather(0, 0)
        fire_gather(1, 1)

        def process(g, bi):
            wait_gather(bi)
            start = lax.rem(base + g * _CH, S)
            buf = bufs[bi]

            @plsc.parallel_loop(0, _CH, 1, unroll=8)
            def radd(r):
                p = start + r
                for k in range(D // 16):
                    sl = pl.ds(k * 16, 16)
                    buf[r, sl] = buf[r, sl] + pe_v[p, sl]

            pltpu.async_copy(buf, out_hbm.at[pl.ds(base + g * _CH, _CH)], ssems[bi])

            nb = (bi + 2) % _NBUF

            @pl.when(g >= 2)
            def _():
                wait_store(nb)

            @pl.when(g + 2 < n_chunks)
            def _():
                fire_gather(g + 2, nb)

        def group(t, _):
            for b in range(_NBUF):
                process(t * _NBUF + b, b)
            return 0

        lax.fori_loop(0, n_chunks // _NBUF, group, 0)
        wait_store(_NBUF - 2)
        wait_store(_NBUF - 1)

    return emb(tok_table, idx, pe_ext)


def kernel(x, tok_table):
    B, S = x.shape
    V, D = tok_table.shape
    N = B * S
    idx = x.reshape(N).astype(jnp.int32)
    pe = _pe_table(S, D)
    pe_ext = jnp.concatenate([pe, pe[:_CH]], axis=0)
    out = _emb_call(tok_table, idx, pe_ext, N, D, S)
    return out.reshape(B, S, D)


# speedup vs baseline: 1.8190x; 1.2612x over previous
import functools

import jax
import jax.numpy as jnp
from jax import lax
from jax.experimental import pallas as pl
from jax.experimental.pallas import tpu as pltpu
from jax.experimental.pallas import tpu_sc as plsc

_NC = 2
_NS = 16
_NW = _NC * _NS
_CH = 128


def _pe_table(max_len, d_embed):
    pos = jnp.arange(max_len, dtype=jnp.float32)[:, None]
    i = jnp.arange(0, d_embed, 2, dtype=jnp.float32)[None, :]
    angle = pos / jnp.power(10000.0, i / d_embed)
    pe = jnp.zeros((max_len, d_embed), dtype=jnp.float32)
    pe = pe.at[:, 0::2].set(jnp.sin(angle))
    pe = pe.at[:, 1::2].set(jnp.cos(angle))
    return pe


_NBUF = 4


_DP = 128


@functools.partial(jax.jit, static_argnums=(3, 4, 5))
def _emb_call(tok_pad, idx, pe_ext, N, D, S):
    n_per_w = N // _NW
    n_chunks = n_per_w // _CH
    assert n_chunks % _NBUF == 0
    pe_rows = pe_ext.shape[0]
    mesh = plsc.VectorSubcoreMesh(core_axis_name="c", subcore_axis_name="s")

    @functools.partial(
        pl.kernel,
        out_type=jax.ShapeDtypeStruct((N, _DP), jnp.float32),
        mesh=mesh,
        scratch_types=[
            pltpu.VMEM((n_per_w,), jnp.int32),
            [pltpu.VMEM((_CH, _DP), jnp.float32)] * _NBUF,
            pltpu.VMEM((pe_rows, D), jnp.float32),
            [pltpu.SemaphoreType.DMA] * _NBUF,
            [pltpu.SemaphoreType.DMA] * _NBUF,
        ],
        compiler_params=pltpu.CompilerParams(use_tc_tiling_on_sc=False),
    )
    def emb(tok_hbm, idx_hbm, pe_hbm, out_hbm, idx_v, bufs, pe_v, gsems, ssems):
        wid = lax.axis_index("s") * _NC + lax.axis_index("c")
        base = wid * n_per_w
        pltpu.sync_copy(idx_hbm.at[pl.ds(base, n_per_w)], idx_v)
        pltpu.sync_copy(pe_hbm, pe_v)

        def fire_gather(g, bi):
            pltpu.async_copy(
                tok_hbm.at[idx_v.at[pl.ds(g * _CH, _CH)]], bufs[bi], gsems[bi]
            )

        def wait_gather(bi):
            pltpu.make_async_copy(
                tok_hbm.at[idx_v.at[pl.ds(0, _CH)]], bufs[bi], gsems[bi]
            ).wait()

        def wait_store(bi):
            pltpu.make_async_copy(
                bufs[bi], out_hbm.at[pl.ds(0, _CH)], ssems[bi]
            ).wait()

        fire_gather(0, 0)
        fire_gather(1, 1)

        def process(g, bi):
            wait_gather(bi)
            start = lax.rem(base + g * _CH, S)
            buf = bufs[bi]

            @plsc.parallel_loop(0, _CH, 1, unroll=8)
            def radd(r):
                p = start + r
                for k in range(D // 16):
                    sl = pl.ds(k * 16, 16)
                    buf[r, sl] = buf[r, sl] + pe_v[p, sl]

            pltpu.async_copy(buf, out_hbm.at[pl.ds(base + g * _CH, _CH)], ssems[bi])

            nb = (bi + 2) % _NBUF

            @pl.when(g >= 2)
            def _():
                wait_store(nb)

            @pl.when(g + 2 < n_chunks)
            def _():
                fire_gather(g + 2, nb)

        def group(t, _):
            for b in range(_NBUF):
                process(t * _NBUF + b, b)
            return 0

        lax.fori_loop(0, n_chunks // _NBUF, group, 0)
        wait_store(_NBUF - 2)
        wait_store(_NBUF - 1)

    return emb(tok_pad, idx, pe_ext)


def kernel(x, tok_table):
    B, S = x.shape
    V, D = tok_table.shape
    N = B * S
    idx = x.reshape(N).astype(jnp.int32)
    pe = _pe_table(S, D)
    pe_ext = jnp.concatenate([pe, pe[:_CH]], axis=0)
    tok_pad = jnp.pad(tok_table, ((0, 0), (0, _DP - D)))
    out = _emb_call(tok_pad, idx, pe_ext, N, D, S)
    return out.reshape(B, S, _DP)[:, :, :D]
